# baseline (speedup 1.0000x reference)
import functools

import jax
import jax.numpy as jnp
from jax import lax
from jax.experimental import pallas as pl
from jax.experimental.pallas import tpu as pltpu
from jax.experimental.pallas import tpu_sc as plsc

_SC_NODES = 4096
_TC_BLOCK_N = 656


def _mean_body(x_ref, o_ref, *, inv_w):
    o_ref[...] = jnp.sum(x_ref[...], axis=1) * inv_w


def _tc_time_mean(final_emb, n_tc, block_n):
    V, W, D = final_emb.shape
    grid = (n_tc // block_n,)
    return pl.pallas_call(
        functools.partial(_mean_body, inv_w=1.0 / W),
        grid=grid,
        in_specs=[pl.BlockSpec((block_n, W, D), lambda i: (i, 0, 0))],
        out_specs=pl.BlockSpec((block_n, D), lambda i: (i, 0)),
        out_shape=jax.ShapeDtypeStruct((n_tc, D), jnp.float32),
    )(final_emb)


def _make_sc_reduce(V, W, D, start):
    info = plsc.get_sparse_core_info()
    NC, NS = info.num_cores, info.num_subcores
    NW = NC * NS
    n_nodes = V - start
    per_w = n_nodes // NW
    CN = 4
    n_chunks = per_w // CN
    mesh = plsc.VectorSubcoreMesh(core_axis_name="c", subcore_axis_name="s")

    @functools.partial(
        pl.kernel,
        mesh=mesh,
        out_type=jax.ShapeDtypeStruct((n_nodes, D), jnp.float32),
        scratch_types=[
            pltpu.VMEM((2, CN, W, D), jnp.float32),
            pltpu.VMEM((per_w, D), jnp.float32),
            pltpu.SemaphoreType.DMA,
            pltpu.SemaphoreType.DMA,
        ],
    )
    def reduce(emb_hbm, out_hbm, buf, rows, sem0, sem1):
        wid = lax.axis_index("s") * NC + lax.axis_index("c")
        base = start + wid * per_w
        sems = (sem0, sem1)
        inv_w = jnp.float32(1.0 / W)

        def chunk_src(c):
            return emb_hbm.at[pl.ds(base + c * CN, CN)]

        pltpu.async_copy(chunk_src(0), buf.at[0], sems[0])
        pltpu.async_copy(chunk_src(1), buf.at[1], sems[1])

        def body(g, _):
            for b in range(2):
                c = g * 2 + b
                pltpu.make_async_copy(chunk_src(c), buf.at[b], sems[b]).wait()
                for n in range(CN):
                    row = c * CN + n
                    for d8 in range(D // 16):
                        sl = pl.ds(d8 * 16, 16)
                        acc = buf[b, n, 0, sl]
                        for t in range(1, W):
                            acc = acc + buf[b, n, t, sl]
                        rows[row, sl] = acc * inv_w

                @pl.when(c + 2 < n_chunks)
                def _():
                    pltpu.async_copy(chunk_src(c + 2), buf.at[b], sems[b])
            return _

        lax.fori_loop(0, n_chunks // 2, body, None)
        pltpu.sync_copy(rows, out_hbm.at[pl.ds(wid * per_w, per_w)])

    return reduce


def _make_sc_gather(V, D, B):
    info = plsc.get_sparse_core_info()
    NC, NS = info.num_cores, info.num_subcores
    NW = NC * NS
    b_per_w = B // NW
    CHUNK = 128
    n_chunks = b_per_w // CHUNK
    mesh = plsc.VectorSubcoreMesh(core_axis_name="c", subcore_axis_name="s")

    @functools.partial(
        pl.kernel,
        mesh=mesh,
        out_type=jax.ShapeDtypeStruct((B, D), jnp.float32),
        scratch_types=[
            pltpu.VMEM((n_chunks, CHUNK), jnp.int32),
            pltpu.VMEM((b_per_w, D), jnp.float32),
            pltpu.SemaphoreType.DMA,
        ],
    )
    def gather(table_hbm, idx_hbm, out_hbm, idx_v, rows_v, sem):
        wid = lax.axis_index("s") * NC + lax.axis_index("c")
        pltpu.sync_copy(idx_hbm.at[pl.ds(wid * n_chunks, n_chunks)], idx_v)
        copies = []
        for j in range(n_chunks):
            copies.append(pltpu.async_copy(
                table_hbm.at[idx_v.at[j]],
                rows_v.at[pl.ds(j * CHUNK, CHUNK)],
                sem,
            ))
        for c in copies:
            c.wait()
        pltpu.sync_copy(rows_v, out_hbm.at[pl.ds(wid * b_per_w, b_per_w)])

    return gather


def kernel(final_emb, node, time):
    V, W, D = final_emb.shape
    B = node.shape[0]
    n_tc = V - _SC_NODES
    table_tc = _tc_time_mean(final_emb, n_tc, _TC_BLOCK_N)
    table_sc = _make_sc_reduce(V, W, D, n_tc)(final_emb)
    table = jnp.concatenate([table_tc, table_sc], axis=0)
    idx = node.reshape(B // 128, 128).astype(jnp.int32)
    rows = _make_sc_gather(V, D, B)(table, idx)
    return rows.reshape(B, 1, D)


# speedup vs baseline: 1.2327x; 1.2327x over previous
import functools

import jax
import jax.numpy as jnp
from jax import lax
from jax.experimental import pallas as pl
from jax.experimental.pallas import tpu as pltpu
from jax.experimental.pallas import tpu_sc as plsc

_SC_NODES = 4096
_TC_BLOCK_N = 656


def _mean_body(x_ref, o_ref, *, inv_w):
    o_ref[...] = jnp.sum(x_ref[...], axis=1) * inv_w


def _tc_time_mean(final_emb, n_tc, block_n):
    V, W, D = final_emb.shape
    grid = (n_tc // block_n,)
    return pl.pallas_call(
        functools.partial(_mean_body, inv_w=1.0 / W),
        grid=grid,
        in_specs=[pl.BlockSpec((block_n, W, D), lambda i: (i, 0, 0))],
        out_specs=pl.BlockSpec((block_n, D), lambda i: (i, 0)),
        out_shape=jax.ShapeDtypeStruct((n_tc, D), jnp.float32),
    )(final_emb)


def _make_sc_reduce(V, W, D, start):
    info = plsc.get_sparse_core_info()
    NC, NS = info.num_cores, info.num_subcores
    NW = NC * NS
    n_nodes = V - start
    per_w = n_nodes // NW
    CN = 8
    n_chunks = per_w // CN
    mesh = plsc.VectorSubcoreMesh(core_axis_name="c", subcore_axis_name="s")

    @functools.partial(
        pl.kernel,
        mesh=mesh,
        out_type=jax.ShapeDtypeStruct((n_nodes, D), jnp.float32),
        scratch_types=[
            pltpu.VMEM((2, CN, W, D), jnp.float32),
            pltpu.VMEM((per_w, D), jnp.float32),
            pltpu.SemaphoreType.DMA,
            pltpu.SemaphoreType.DMA,
        ],
    )
    def reduce(emb_hbm, out_hbm, buf, rows, sem0, sem1):
        wid = lax.axis_index("s") * NC + lax.axis_index("c")
        base = start + wid * per_w
        sems = (sem0, sem1)
        inv_w = jnp.float32(1.0 / W)

        def chunk_src(c):
            return emb_hbm.at[pl.ds(base + c * CN, CN)]

        pltpu.async_copy(chunk_src(0), buf.at[0], sems[0])
        pltpu.async_copy(chunk_src(1), buf.at[1], sems[1])

        def body(g, _):
            for b in range(2):
                c = g * 2 + b
                pltpu.make_async_copy(chunk_src(c), buf.at[b], sems[b]).wait()
                for n in range(CN):
                    row = c * CN + n
                    for d8 in range(D // 16):
                        sl = pl.ds(d8 * 16, 16)
                        accs = [buf[b, n, t, sl] for t in range(4)]
                        for t in range(4, W):
                            accs[t % 4] = accs[t % 4] + buf[b, n, t, sl]
                        rows[row, sl] = ((accs[0] + accs[1])
                                         + (accs[2] + accs[3])) * inv_w

                @pl.when(c + 2 < n_chunks)
                def _():
                    pltpu.async_copy(chunk_src(c + 2), buf.at[b], sems[b])
            return _

        lax.fori_loop(0, n_chunks // 2, body, None)
        pltpu.sync_copy(rows, out_hbm.at[pl.ds(wid * per_w, per_w)])

    return reduce


def _make_sc_gather(V, D, B):
    info = plsc.get_sparse_core_info()
    NC, NS = info.num_cores, info.num_subcores
    NW = NC * NS
    b_per_w = B // NW
    CHUNK = 128
    n_chunks = b_per_w // CHUNK
    mesh = plsc.VectorSubcoreMesh(core_axis_name="c", subcore_axis_name="s")

    @functools.partial(
        pl.kernel,
        mesh=mesh,
        out_type=jax.ShapeDtypeStruct((B, D), jnp.float32),
        scratch_types=[
            pltpu.VMEM((n_chunks, CHUNK), jnp.int32),
            pltpu.VMEM((b_per_w, D), jnp.float32),
            pltpu.SemaphoreType.DMA,
        ],
    )
    def gather(table_hbm, idx_hbm, out_hbm, idx_v, rows_v, sem):
        wid = lax.axis_index("s") * NC + lax.axis_index("c")
        pltpu.sync_copy(idx_hbm.at[pl.ds(wid * n_chunks, n_chunks)], idx_v)
        copies = []
        for j in range(n_chunks):
            copies.append(pltpu.async_copy(
                table_hbm.at[idx_v.at[j]],
                rows_v.at[pl.ds(j * CHUNK, CHUNK)],
                sem,
            ))
        for c in copies:
            c.wait()
        pltpu.sync_copy(rows_v, out_hbm.at[pl.ds(wid * b_per_w, b_per_w)])

    return gather


def kernel(final_emb, node, time):
    V, W, D = final_emb.shape
    B = node.shape[0]
    n_tc = V - _SC_NODES
    table_tc = _tc_time_mean(final_emb, n_tc, _TC_BLOCK_N)
    table_sc = _make_sc_reduce(V, W, D, n_tc)(final_emb)
    table = jnp.concatenate([table_tc, table_sc], axis=0)
    idx = node.reshape(B // 128, 128).astype(jnp.int32)
    rows = _make_sc_gather(V, D, B)(table, idx)
    return rows.reshape(B, 1, D)


# speedup vs baseline: 1.2623x; 1.0240x over previous
import functools

import jax
import jax.numpy as jnp
from jax import lax
from jax.experimental import pallas as pl
from jax.experimental.pallas import tpu as pltpu
from jax.experimental.pallas import tpu_sc as plsc

_SC_NODES = 4096
_TC_BLOCK_N = 656


def _mean_body(x_ref, o_ref, *, inv_w):
    o_ref[...] = jnp.sum(x_ref[...], axis=1) * inv_w


def _tc_time_mean(final_emb, n_tc, block_n):
    V, W, D = final_emb.shape
    grid = (n_tc // block_n,)
    return pl.pallas_call(
        functools.partial(_mean_body, inv_w=1.0 / W),
        grid=grid,
        in_specs=[pl.BlockSpec((block_n, W, D), lambda i: (i, 0, 0))],
        out_specs=pl.BlockSpec((block_n, D), lambda i: (i, 0)),
        out_shape=jax.ShapeDtypeStruct((n_tc, D), jnp.float32),
    )(final_emb)


def _make_sc_reduce(V, W, D, start):
    info = plsc.get_sparse_core_info()
    NC, NS = info.num_cores, info.num_subcores
    NW = NC * NS
    n_nodes = V - start
    per_w = n_nodes // NW
    CN = 4
    NBUF = 4
    n_chunks = per_w // CN
    mesh = plsc.VectorSubcoreMesh(core_axis_name="c", subcore_axis_name="s")

    @functools.partial(
        pl.kernel,
        mesh=mesh,
        out_type=jax.ShapeDtypeStruct((n_nodes, D), jnp.float32),
        scratch_types=[
            pltpu.VMEM((NBUF, CN, W, D), jnp.float32),
            pltpu.VMEM((per_w, D), jnp.float32),
            pltpu.SemaphoreType.DMA,
            pltpu.SemaphoreType.DMA,
            pltpu.SemaphoreType.DMA,
            pltpu.SemaphoreType.DMA,
        ],
    )
    def reduce(emb_hbm, out_hbm, buf, rows, sem0, sem1, sem2, sem3):
        wid = lax.axis_index("s") * NC + lax.axis_index("c")
        base = start + wid * per_w
        sems = (sem0, sem1, sem2, sem3)
        inv_w = jnp.float32(1.0 / W)

        def chunk_src(c):
            return emb_hbm.at[pl.ds(base + c * CN, CN)]

        for b in range(NBUF):
            pltpu.async_copy(chunk_src(b), buf.at[b], sems[b])

        def body(g, _):
            for b in range(NBUF):
                c = g * NBUF + b
                pltpu.make_async_copy(chunk_src(c), buf.at[b], sems[b]).wait()
                for n in range(CN):
                    row = c * CN + n
                    for d8 in range(D // 16):
                        sl = pl.ds(d8 * 16, 16)
                        accs = [buf[b, n, t, sl] for t in range(4)]
                        for t in range(4, W):
                            accs[t % 4] = accs[t % 4] + buf[b, n, t, sl]
                        rows[row, sl] = ((accs[0] + accs[1])
                                         + (accs[2] + accs[3])) * inv_w

                @pl.when(c + NBUF < n_chunks)
                def _():
                    pltpu.async_copy(chunk_src(c + NBUF), buf.at[b], sems[b])
            return _

        lax.fori_loop(0, n_chunks // NBUF, body, None)
        pltpu.sync_copy(rows, out_hbm.at[pl.ds(wid * per_w, per_w)])

    return reduce


def _make_sc_gather(V, D, B):
    info = plsc.get_sparse_core_info()
    NC, NS = info.num_cores, info.num_subcores
    NW = NC * NS
    b_per_w = B // NW
    CHUNK = 128
    n_chunks = b_per_w // CHUNK
    mesh = plsc.VectorSubcoreMesh(core_axis_name="c", subcore_axis_name="s")

    @functools.partial(
        pl.kernel,
        mesh=mesh,
        out_type=jax.ShapeDtypeStruct((B, D), jnp.float32),
        scratch_types=[
            pltpu.VMEM((n_chunks, CHUNK), jnp.int32),
            pltpu.VMEM((b_per_w, D), jnp.float32),
            pltpu.SemaphoreType.DMA,
        ],
    )
    def gather(table_hbm, idx_hbm, out_hbm, idx_v, rows_v, sem):
        wid = lax.axis_index("s") * NC + lax.axis_index("c")
        pltpu.sync_copy(idx_hbm.at[pl.ds(wid * n_chunks, n_chunks)], idx_v)
        copies = []
        for j in range(n_chunks):
            copies.append(pltpu.async_copy(
                table_hbm.at[idx_v.at[j]],
                rows_v.at[pl.ds(j * CHUNK, CHUNK)],
                sem,
            ))
        for c in copies:
            c.wait()
        pltpu.sync_copy(rows_v, out_hbm.at[pl.ds(wid * b_per_w, b_per_w)])

    return gather


def kernel(final_emb, node, time):
    V, W, D = final_emb.shape
    B = node.shape[0]
    n_tc = V - _SC_NODES
    table_tc = _tc_time_mean(final_emb, n_tc, _TC_BLOCK_N)
    table_sc = _make_sc_reduce(V, W, D, n_tc)(final_emb)
    table = jnp.concatenate([table_tc, table_sc], axis=0)
    idx = node.reshape(B // 128, 128).astype(jnp.int32)
    rows = _make_sc_gather(V, D, B)(table, idx)
    return rows.reshape(B, 1, D)


# speedup vs baseline: 1.8374x; 1.4556x over previous
import functools

import jax
import jax.numpy as jnp
from jax import lax
from jax.experimental import pallas as pl
from jax.experimental.pallas import tpu as pltpu
from jax.experimental.pallas import tpu_sc as plsc

_SC_NODES = 2048
_TC_BLOCK_N = 568


def _mean_body(x_ref, o_ref, *, inv_w):
    o_ref[...] = jnp.sum(x_ref[...], axis=1) * inv_w


def _tc_time_mean(final_emb, n_tc, block_n):
    V, W, D = final_emb.shape
    grid = (n_tc // block_n,)
    return pl.pallas_call(
        functools.partial(_mean_body, inv_w=1.0 / W),
        grid=grid,
        in_specs=[pl.BlockSpec((block_n, W, D), lambda i: (i, 0, 0))],
        out_specs=pl.BlockSpec((block_n, D), lambda i: (i, 0)),
        out_shape=jax.ShapeDtypeStruct((n_tc, D), jnp.float32),
    )(final_emb)


def _make_sc_reduce(V, W, D, start):
    info = plsc.get_sparse_core_info()
    NC, NS = info.num_cores, info.num_subcores
    NW = NC * NS
    n_nodes = V - start
    per_w = n_nodes // NW
    CN = 4
    NBUF = 4
    n_chunks = per_w // CN
    RPC = CN * W
    mesh = plsc.VectorSubcoreMesh(core_axis_name="c", subcore_axis_name="s")

    @functools.partial(
        pl.kernel,
        mesh=mesh,
        out_type=jax.ShapeDtypeStruct((n_nodes, D), jnp.float32),
        scratch_types=[
            pltpu.VMEM((NBUF, RPC, D), jnp.float32),
            pltpu.VMEM((per_w, D), jnp.float32),
            pltpu.SemaphoreType.DMA,
            pltpu.SemaphoreType.DMA,
            pltpu.SemaphoreType.DMA,
            pltpu.SemaphoreType.DMA,
        ],
    )
    def reduce(emb_hbm, out_hbm, buf, rows, sem0, sem1, sem2, sem3):
        wid = lax.axis_index("s") * NC + lax.axis_index("c")
        base = (start + wid * per_w) * W
        sems = (sem0, sem1, sem2, sem3)
        inv_w = jnp.float32(1.0 / W)

        def chunk_copy(c, b):
            return pltpu.make_async_copy(
                emb_hbm.at[pl.ds(base + c * RPC, RPC)],
                buf.at[b],
                sems[b],
            )

        for b in range(NBUF):
            chunk_copy(b, b).start()

        def body(g, _):
            for b in range(NBUF):
                c = g * NBUF + b
                chunk_copy(c, b).wait()
                for n in range(CN):
                    row = c * CN + n
                    for d8 in range(D // 16):
                        sl = pl.ds(d8 * 16, 16)
                        accs = [buf[b, n * W + t, sl] for t in range(4)]
                        for t in range(4, W):
                            accs[t % 4] = accs[t % 4] + buf[b, n * W + t, sl]
                        rows[row, sl] = ((accs[0] + accs[1])
                                         + (accs[2] + accs[3])) * inv_w

                @pl.when(c + NBUF < n_chunks)
                def _():
                    chunk_copy(c + NBUF, b).start()
            return _

        lax.fori_loop(0, n_chunks // NBUF, body, None)
        pltpu.sync_copy(rows, out_hbm.at[pl.ds(wid * per_w, per_w)])

    return reduce


def _make_sc_gather(V, D, B):
    info = plsc.get_sparse_core_info()
    NC, NS = info.num_cores, info.num_subcores
    NW = NC * NS
    b_per_w = B // NW
    CHUNK = 128
    n_chunks = b_per_w // CHUNK
    mesh = plsc.VectorSubcoreMesh(core_axis_name="c", subcore_axis_name="s")

    @functools.partial(
        pl.kernel,
        mesh=mesh,
        out_type=jax.ShapeDtypeStruct((B, D), jnp.float32),
        scratch_types=[
            pltpu.VMEM((n_chunks, CHUNK), jnp.int32),
            pltpu.VMEM((b_per_w, D), jnp.float32),
            pltpu.SemaphoreType.DMA,
        ],
    )
    def gather(table_hbm, idx_hbm, out_hbm, idx_v, rows_v, sem):
        wid = lax.axis_index("s") * NC + lax.axis_index("c")
        pltpu.sync_copy(idx_hbm.at[pl.ds(wid * n_chunks, n_chunks)], idx_v)
        copies = []
        for j in range(n_chunks):
            copies.append(pltpu.async_copy(
                table_hbm.at[idx_v.at[j]],
                rows_v.at[pl.ds(j * CHUNK, CHUNK)],
                sem,
            ))
        for c in copies:
            c.wait()
        pltpu.sync_copy(rows_v, out_hbm.at[pl.ds(wid * b_per_w, b_per_w)])

    return gather


def kernel(final_emb, node, time):
    V, W, D = final_emb.shape
    B = node.shape[0]
    n_tc = V - _SC_NODES
    table_tc = _tc_time_mean(final_emb, n_tc, _TC_BLOCK_N)
    table_sc = _make_sc_reduce(V, W, D, n_tc)(final_emb.reshape(V * W, D))
    table = jnp.concatenate([table_tc, table_sc], axis=0)
    idx = node.reshape(B // 128, 128).astype(jnp.int32)
    rows = _make_sc_gather(V, D, B)(table, idx)
    return rows.reshape(B, 1, D)


# speedup vs baseline: 2.1526x; 1.1715x over previous
import functools

import jax
import jax.numpy as jnp
from jax import lax
from jax.experimental import pallas as pl
from jax.experimental.pallas import tpu as pltpu
from jax.experimental.pallas import tpu_sc as plsc


def _mean_body(x_ref, o_ref, *, inv_w):
    o_ref[...] = jnp.sum(x_ref[...], axis=1) * inv_w


@functools.partial(jax.jit, static_argnames=("block_n",))
def _time_mean(final_emb, block_n=400):
    V, W, D = final_emb.shape
    grid = (V // block_n,)
    return pl.pallas_call(
        functools.partial(_mean_body, inv_w=1.0 / W),
        grid=grid,
        in_specs=[pl.BlockSpec((block_n, W, D), lambda i: (i, 0, 0))],
        out_specs=pl.BlockSpec((block_n, D), lambda i: (i, 0)),
        out_shape=jax.ShapeDtypeStruct((V, D), jnp.float32),
    )(final_emb)


def _make_sc_gather(V, D, B):
    info = plsc.get_sparse_core_info()
    NC, NS = info.num_cores, info.num_subcores
    NW = NC * NS
    b_per_w = B // NW
    CHUNK = 128
    n_chunks = b_per_w // CHUNK
    mesh = plsc.VectorSubcoreMesh(core_axis_name="c", subcore_axis_name="s")

    @functools.partial(
        pl.kernel,
        mesh=mesh,
        out_type=jax.ShapeDtypeStruct((B, D), jnp.float32),
        scratch_types=[
            pltpu.VMEM((n_chunks, CHUNK), jnp.int32),
            pltpu.VMEM((b_per_w, D), jnp.float32),
            pltpu.SemaphoreType.DMA,
            pltpu.SemaphoreType.DMA,
            pltpu.SemaphoreType.DMA,
            pltpu.SemaphoreType.DMA,
            pltpu.SemaphoreType.DMA,
        ],
    )
    def gather(table_hbm, idx_hbm, out_hbm, idx_v, rows_v,
               g0, g1, g2, g3, wsem):
        wid = lax.axis_index("s") * NC + lax.axis_index("c")
        gsems = (g0, g1, g2, g3)
        pltpu.sync_copy(idx_hbm.at[pl.ds(wid * n_chunks, n_chunks)], idx_v)
        gathers = []
        for j in range(n_chunks):
            gathers.append(pltpu.async_copy(
                table_hbm.at[idx_v.at[j]],
                rows_v.at[pl.ds(j * CHUNK, CHUNK)],
                gsems[j],
            ))
        writes = []
        for j in range(n_chunks):
            gathers[j].wait()
            writes.append(pltpu.async_copy(
                rows_v.at[pl.ds(j * CHUNK, CHUNK)],
                out_hbm.at[pl.ds(wid * b_per_w + j * CHUNK, CHUNK)],
                wsem,
            ))
        for wr in writes:
            wr.wait()

    return gather


def kernel(final_emb, node, time):
    V, W, D = final_emb.shape
    B = node.shape[0]
    table = _time_mean(final_emb)
    idx = node.reshape(B // 128, 128).astype(jnp.int32)
    rows = _make_sc_gather(V, D, B)(table, idx)
    return rows.reshape(B, 1, D)
